# baseline (speedup 1.0000x reference)
import functools

import jax
import jax.numpy as jnp
from jax.experimental import pallas as pl
from jax.experimental.pallas import tpu as pltpu

_HIGHEST = jax.lax.Precision.HIGHEST


def _router_body(x_ref, rw1_ref, rb1_ref, rw2_ref, rb2_ref, w_ref):
    n = rw2_ref.shape[-1]
    x = x_ref[...].astype(jnp.bfloat16)
    h = jnp.dot(x, rw1_ref[...].astype(jnp.bfloat16),
                preferred_element_type=jnp.float32) + rb1_ref[...]
    h = jnp.maximum(h, 0.0).astype(jnp.bfloat16)
    s = jnp.dot(h, rw2_ref[...].astype(jnp.bfloat16),
                preferred_element_type=jnp.float32) + rb2_ref[...]
    lane = jax.lax.broadcasted_iota(jnp.int32, s.shape, 1)
    m1 = jnp.max(s, axis=1, keepdims=True)
    a1 = jnp.min(jnp.where(s == m1, lane, n), axis=1, keepdims=True)
    s_masked = jnp.where(lane == a1, -jnp.inf, s)
    m2 = jnp.max(s_masked, axis=1, keepdims=True)
    a2 = jnp.min(jnp.where(s_masked == m2, lane, n), axis=1, keepdims=True)
    e2 = jnp.exp(m2 - m1)
    w1 = 1.0 / (1.0 + e2)
    w2 = e2 / (1.0 + e2)
    w_ref[...] = jnp.where(lane == a1, w1, 0.0) + jnp.where(lane == a2, w2, 0.0)


def _expert_body(w_ref, x_ref, ew1_ref, eb1_ref, ew2_ref, eb2_ref, o_ref):
    e = pl.program_id(1)
    m = x_ref.shape[0]
    x = x_ref[...].astype(jnp.bfloat16)
    h = jnp.dot(x, ew1_ref[0], preferred_element_type=jnp.float32) + eb1_ref[0]
    h = jnp.maximum(h, 0.0).astype(jnp.bfloat16)
    y = jnp.dot(h, ew2_ref[0], preferred_element_type=jnp.float32) + eb2_ref[0]
    w = w_ref[...]
    lane = jax.lax.broadcasted_iota(jnp.int32, w.shape, 1)
    wcol = jnp.sum(jnp.where(lane == e, w, 0.0), axis=1, keepdims=True)
    contrib = y * wcol

    @pl.when(e == 0)
    def _():
        o_ref[...] = contrib

    @pl.when(e > 0)
    def _():
        o_ref[...] += contrib


@jax.jit
def kernel(inputs, rw1, rb1, rw2, rb2, ew1, eb1, ew2, eb2):
    b, s, e = inputs.shape
    n = rw2.shape[-1]
    h = ew1.shape[-1]
    x = inputs.reshape(s, e)

    blk = 256
    weights = pl.pallas_call(
        _router_body,
        grid=(s // blk,),
        in_specs=[
            pl.BlockSpec((blk, e), lambda i: (i, 0)),
            pl.BlockSpec((e, e), lambda i: (0, 0)),
            pl.BlockSpec((e,), lambda i: (0,)),
            pl.BlockSpec((e, n), lambda i: (0, 0)),
            pl.BlockSpec((n,), lambda i: (0,)),
        ],
        out_specs=pl.BlockSpec((blk, n), lambda i: (i, 0)),
        out_shape=jax.ShapeDtypeStruct((s, n), jnp.float32),
    )(x, rw1, rb1, rw2, rb2)

    ew1_bf = ew1.astype(jnp.bfloat16)
    ew2_bf = ew2.astype(jnp.bfloat16)
    eb1_3d = eb1.reshape(n, 1, h)
    eb2_3d = eb2.reshape(n, 1, e)

    mblk = 256
    out = pl.pallas_call(
        _expert_body,
        grid=(s // mblk, n),
        in_specs=[
            pl.BlockSpec((mblk, n), lambda i, j: (i, 0)),
            pl.BlockSpec((mblk, e), lambda i, j: (i, 0)),
            pl.BlockSpec((1, e, h), lambda i, j: (j, 0, 0)),
            pl.BlockSpec((1, 1, h), lambda i, j: (j, 0, 0)),
            pl.BlockSpec((1, h, e), lambda i, j: (j, 0, 0)),
            pl.BlockSpec((1, 1, e), lambda i, j: (j, 0, 0)),
        ],
        out_specs=pl.BlockSpec((mblk, e), lambda i, j: (i, 0)),
        out_shape=jax.ShapeDtypeStruct((s, e), jnp.float32),
        compiler_params=pltpu.CompilerParams(
            dimension_semantics=("parallel", "arbitrary"),
        ),
    )(weights, x, ew1_bf, eb1_3d, ew2_bf, eb2_3d)

    return out.reshape(b, s, e)
